# baseline (speedup 1.0000x reference)
import jax
import jax.numpy as jnp
from jax import lax
from jax.experimental import pallas as pl
from jax.experimental.pallas import tpu as pltpu
from jax.experimental.pallas import tpu_sc as plsc

_K = 1024
_D = 64
_N = 9216
_NW = 32
_CHUNK = 96
_NCH = (_N // _NW) // _CHUNK
_LANES = 16


def _dist_argmin_kernel(xt_ref, cbt_ref, idx_ref, loss_ref, esum_ref,
                        perp_ref):
    xt = xt_ref[...]
    xsq = jnp.sum(xt * xt, axis=0, keepdims=True)
    cbt = cbt_ref[...]
    scores_t = lax.dot_general(
        cbt, xt, (((0,), (0,)), ((), ())),
        preferred_element_type=jnp.float32)
    d = jnp.sum(cbt * cbt, axis=0)[:, None] - 2.0 * scores_t
    idx = jnp.argmin(d, axis=0).astype(jnp.int32)
    idx_ref[...] = idx
    loss_ref[...] = ((jnp.sum(jnp.min(d, axis=0)) + jnp.sum(xsq))
                     * (1.25 / (_N * _D)))[None, None]
    bins = lax.broadcasted_iota(jnp.int32, (_K, _N), 0)
    onehot = jnp.where(bins == idx[None, :], 1.0, 0.0)
    h = jnp.sum(onehot, axis=1)
    esum_ref[...] = h
    p = h * (1.0 / _N)
    ent = jnp.sum(p * jnp.log(p + 1e-10))
    perp_ref[...] = jnp.exp(-ent)[None, None]


def _sc_gather_kernel(cb_hbm, idx_hbm, q_hbm, idx_v, rows_v, sem):
    wid = lax.axis_index("s") * 2 + lax.axis_index("c")
    base = wid * _NCH
    pltpu.sync_copy(idx_hbm.at[wid], idx_v)
    copies = [
        pltpu.async_copy(cb_hbm.at[idx_v.at[j]], rows_v.at[j], sem)
        for j in range(_NCH)
    ]
    for cp in copies:
        cp.wait()
    pltpu.sync_copy(rows_v, q_hbm.at[pl.ds(base, _NCH)])


def _hist_entropy_kernel(idx_ref, esum_ref, perp_ref):
    idx = idx_ref[...]
    bins = lax.broadcasted_iota(jnp.int32, (_K, _N), 0)
    onehot = jnp.where(bins == idx[None, :], 1.0, 0.0)
    h = jnp.sum(onehot, axis=1)
    esum_ref[...] = h
    p = h * (1.0 / _N)
    ent = jnp.sum(p * jnp.log(p + 1e-10))
    perp_ref[...] = jnp.exp(-ent)[None, None]


def kernel(inputs, codebook):
    B, T, D = inputs.shape
    flat = inputs.reshape(-1, D)

    idx, loss2, esum, perp2 = pl.pallas_call(
        _dist_argmin_kernel,
        grid=(1,),
        in_specs=[pl.BlockSpec((_D, _N), lambda i: (0, 0)),
                  pl.BlockSpec((_D, _K), lambda i: (0, 0))],
        out_specs=[pl.BlockSpec((_N,), lambda i: (0,)),
                   pl.BlockSpec((1, 1), lambda i: (0, 0)),
                   pl.BlockSpec((_K,), lambda i: (0,)),
                   pl.BlockSpec((1, 1), lambda i: (0, 0))],
        out_shape=[jax.ShapeDtypeStruct((_N,), jnp.int32),
                   jax.ShapeDtypeStruct((1, 1), jnp.float32),
                   jax.ShapeDtypeStruct((_K,), jnp.float32),
                   jax.ShapeDtypeStruct((1, 1), jnp.float32)],
    )(flat.T, codebook.T)

    sc = pl.kernel(
        _sc_gather_kernel,
        (jax.ShapeDtypeStruct((_NW * _NCH, _CHUNK, _D), jnp.float32),),
        mesh=plsc.VectorSubcoreMesh(core_axis_name="c", subcore_axis_name="s"),
        compiler_params=pltpu.CompilerParams(needs_layout_passes=False,
                                             use_tc_tiling_on_sc=False),
        scratch_types=[pltpu.VMEM((_NCH, _CHUNK), jnp.int32),
                       pltpu.VMEM((_NCH, _CHUNK, _D), jnp.float32),
                       pltpu.SemaphoreType.DMA],
    )
    (q3,) = sc(codebook, idx.reshape(_NW, _NCH, _CHUNK))
    quantized = q3.reshape(B, T, D)

    return (loss2[0, 0], quantized, esum, codebook, idx, perp2[0, 0])


# speedup vs baseline: 1.0780x; 1.0780x over previous
import jax
import jax.numpy as jnp
from jax import lax
from jax.experimental import pallas as pl
from jax.experimental.pallas import tpu as pltpu
from jax.experimental.pallas import tpu_sc as plsc

_K = 1024
_D = 64
_N = 9216
_NW = 32
_CHUNK = 96
_NCH = (_N // _NW) // _CHUNK
_LANES = 16


def _dist_argmin_kernel(xt_ref, cbt_ref, idx_ref, loss_ref):
    xt = xt_ref[...]
    xsq = jnp.sum(xt * xt, axis=0, keepdims=True)
    cbt = cbt_ref[...]
    scores_t = lax.dot_general(
        cbt, xt, (((0,), (0,)), ((), ())),
        preferred_element_type=jnp.float32)
    d = jnp.sum(cbt * cbt, axis=0)[:, None] - 2.0 * scores_t
    idx_ref[...] = jnp.argmin(d, axis=0).astype(jnp.int32)
    loss_ref[...] = ((jnp.sum(jnp.min(d, axis=0)) + jnp.sum(xsq))
                     * (1.25 / (_N * _D)))[None, None]


def _sc_gather_kernel(cb_hbm, idx_hbm, q_hbm, idx_v, rows_v, sem):
    wid = lax.axis_index("s") * 2 + lax.axis_index("c")
    base = wid * _NCH
    pltpu.sync_copy(idx_hbm.at[wid], idx_v)
    copies = [
        pltpu.async_copy(cb_hbm.at[idx_v.at[j]], rows_v.at[j], sem)
        for j in range(_NCH)
    ]
    for cp in copies:
        cp.wait()
    pltpu.sync_copy(rows_v, q_hbm.at[pl.ds(base, _NCH)])


def _hist_entropy_kernel(idx_ref, esum_ref, perp_ref):
    idx = idx_ref[...]
    bins = lax.broadcasted_iota(jnp.int32, (_K, _N), 0)
    onehot = jnp.where(bins == idx[None, :], 1.0, 0.0)
    h = jnp.sum(onehot, axis=1)
    esum_ref[...] = h
    p = h * (1.0 / _N)
    ent = jnp.sum(p * jnp.log(p + 1e-10))
    perp_ref[...] = jnp.exp(-ent)[None, None]


def kernel(inputs, codebook):
    B, T, D = inputs.shape
    flat = inputs.reshape(-1, D)

    idx, loss2 = pl.pallas_call(
        _dist_argmin_kernel,
        grid=(1,),
        in_specs=[pl.BlockSpec((_D, _N), lambda i: (0, 0)),
                  pl.BlockSpec((_D, _K), lambda i: (0, 0))],
        out_specs=[pl.BlockSpec((_N,), lambda i: (0,)),
                   pl.BlockSpec((1, 1), lambda i: (0, 0))],
        out_shape=[jax.ShapeDtypeStruct((_N,), jnp.int32),
                   jax.ShapeDtypeStruct((1, 1), jnp.float32)],
    )(flat.T, codebook.T)

    sc = pl.kernel(
        _sc_gather_kernel,
        (jax.ShapeDtypeStruct((_NW * _NCH, _CHUNK, _D), jnp.float32),),
        mesh=plsc.VectorSubcoreMesh(core_axis_name="c", subcore_axis_name="s"),
        compiler_params=pltpu.CompilerParams(needs_layout_passes=False,
                                             use_tc_tiling_on_sc=False),
        scratch_types=[pltpu.VMEM((_NCH, _CHUNK), jnp.int32),
                       pltpu.VMEM((_NCH, _CHUNK, _D), jnp.float32),
                       pltpu.SemaphoreType.DMA],
    )
    (q3,) = sc(codebook, idx.reshape(_NW, _NCH, _CHUNK))
    quantized = q3.reshape(B, T, D)

    esum, perp2 = pl.pallas_call(
        _hist_entropy_kernel,
        out_shape=(jax.ShapeDtypeStruct((_K,), jnp.float32),
                   jax.ShapeDtypeStruct((1, 1), jnp.float32)),
    )(idx)

    return (loss2[0, 0], quantized, esum, codebook, idx, perp2[0, 0])
